# baseline (speedup 1.0000x reference)
import functools

import jax
import jax.numpy as jnp
from jax import lax
from jax.experimental import pallas as pl
from jax.experimental.pallas import tpu as pltpu
from jax.experimental.pallas import tpu_sc as plsc

BATCH = 4096
FIELDS = 26
EMBED_DIM = 64
NC = 2
NS = 16
NW = NC * NS
BPW = BATCH // NW
LANES = 16

_mesh = plsc.VectorSubcoreMesh(core_axis_name="c", subcore_axis_name="s")


@functools.partial(
    pl.kernel,
    mesh=_mesh,
    out_type=jax.ShapeDtypeStruct((FIELDS, EMBED_DIM, BATCH), jnp.float32),
    scratch_types=[
        pltpu.VMEM((FIELDS, BPW), jnp.int32),
        pltpu.VMEM((2, BPW, EMBED_DIM), jnp.float32),
        pltpu.VMEM((2, EMBED_DIM, BPW), jnp.float32),
        pltpu.SemaphoreType.DMA,
        pltpu.SemaphoreType.DMA,
        pltpu.SemaphoreType.DMA,
    ],
    compiler_params=pltpu.CompilerParams(
        use_tc_tiling_on_sc=False, needs_layout_passes=False),
)
def _embed_gather(idx_hbm, table_hbm, out_hbm, idx_v, rows_v, tr_v,
                  gsem, ssem0, ssem1):
    ssems = (ssem0, ssem1)
    wid = lax.axis_index("s") * NC + lax.axis_index("c")
    b0 = wid * BPW
    pltpu.sync_copy(idx_hbm.at[:, pl.ds(b0, BPW)], idx_v)

    def start_gather(f, b):
        pltpu.async_copy(table_hbm.at[idx_v.at[f]], rows_v.at[b], gsem)

    start_gather(0, 0)
    start_gather(1, 1)

    def body(p, _):
        for b in range(2):
            f = p * 2 + b
            pltpu.make_async_copy(table_hbm.at[idx_v.at[0]], rows_v.at[b],
                                  gsem).wait()
            @pl.when(p >= 1)
            def _():
                pltpu.make_async_copy(tr_v.at[b],
                                      out_hbm.at[0, :, pl.ds(0, BPW)],
                                      ssems[b]).wait()
            for r0 in range(0, BPW, LANES):
                row16 = lax.iota(jnp.int32, LANES) + r0
                for c in range(EMBED_DIM):
                    cvec = jnp.full((LANES,), c, jnp.int32)
                    col = plsc.load_gather(rows_v.at[b], [row16, cvec])
                    tr_v[b, c, pl.ds(r0, LANES)] = col
            pltpu.async_copy(tr_v.at[b], out_hbm.at[f, :, pl.ds(b0, BPW)],
                             ssems[b])
            @pl.when(f + 2 < FIELDS)
            def _():
                start_gather(f + 2, b)
        return ()

    lax.fori_loop(0, FIELDS // 2, body, (), unroll=False)

    for b in range(2):
        pltpu.make_async_copy(tr_v.at[b], out_hbm.at[0, :, pl.ds(0, BPW)],
                              ssems[b]).wait()


def kernel(feat_ids, table):
    idx = feat_ids.astype(jnp.int32).T
    out = _embed_gather(idx, table)
    return jnp.transpose(out, (2, 0, 1))


# speedup vs baseline: 1.0010x; 1.0010x over previous
import functools

import jax
import jax.numpy as jnp
from jax import lax
from jax.experimental import pallas as pl
from jax.experimental.pallas import tpu as pltpu
from jax.experimental.pallas import tpu_sc as plsc

BATCH = 4096
FIELDS = 26
EMBED_DIM = 64
NC = 2
NS = 16
NW = NC * NS
BPW = BATCH // NW
LANES = 16

_mesh = plsc.VectorSubcoreMesh(core_axis_name="c", subcore_axis_name="s")


@functools.partial(
    pl.kernel,
    mesh=_mesh,
    out_type=jax.ShapeDtypeStruct((FIELDS, EMBED_DIM, BATCH), jnp.float32),
    scratch_types=[
        pltpu.VMEM((FIELDS, BPW), jnp.int32),
        pltpu.VMEM((2, BPW, EMBED_DIM), jnp.float32),
        pltpu.VMEM((2, EMBED_DIM, BPW), jnp.float32),
        pltpu.SemaphoreType.DMA,
        pltpu.SemaphoreType.DMA,
        pltpu.SemaphoreType.DMA,
    ],
    compiler_params=pltpu.CompilerParams(
        use_tc_tiling_on_sc=False, needs_layout_passes=False,
        skip_device_barrier=True, disable_semaphore_checks=True),
)
def _embed_gather(idx_hbm, table_hbm, out_hbm, idx_v, rows_v, tr_v,
                  gsem, ssem0, ssem1):
    ssems = (ssem0, ssem1)
    wid = lax.axis_index("s") * NC + lax.axis_index("c")
    b0 = wid * BPW
    pltpu.sync_copy(idx_hbm.at[:, pl.ds(b0, BPW)], idx_v)

    def start_gather(f, b):
        pltpu.async_copy(table_hbm.at[idx_v.at[f]], rows_v.at[b], gsem)

    start_gather(0, 0)
    start_gather(1, 1)

    def body(p, _):
        for b in range(2):
            f = p * 2 + b
            pltpu.make_async_copy(table_hbm.at[idx_v.at[0]], rows_v.at[b],
                                  gsem).wait()
            @pl.when(p >= 1)
            def _():
                pltpu.make_async_copy(tr_v.at[b],
                                      out_hbm.at[0, :, pl.ds(0, BPW)],
                                      ssems[b]).wait()
            for r0 in range(0, BPW, LANES):
                row16 = lax.iota(jnp.int32, LANES) + r0
                for c in range(EMBED_DIM):
                    cvec = jnp.full((LANES,), c, jnp.int32)
                    col = plsc.load_gather(rows_v.at[b], [row16, cvec])
                    tr_v[b, c, pl.ds(r0, LANES)] = col
            pltpu.async_copy(tr_v.at[b], out_hbm.at[f, :, pl.ds(b0, BPW)],
                             ssems[b])
            @pl.when(f + 2 < FIELDS)
            def _():
                start_gather(f + 2, b)
        return ()

    lax.fori_loop(0, FIELDS // 2, body, (), unroll=False)

    for b in range(2):
        pltpu.make_async_copy(tr_v.at[b], out_hbm.at[0, :, pl.ds(0, BPW)],
                              ssems[b]).wait()


def kernel(feat_ids, table):
    idx = feat_ids.astype(jnp.int32).T
    out = _embed_gather(idx, table)
    return jnp.transpose(out, (2, 0, 1))


# speedup vs baseline: 1.1800x; 1.1788x over previous
import functools

import jax
import jax.numpy as jnp
from jax import lax
from jax.experimental import pallas as pl
from jax.experimental.pallas import tpu as pltpu
from jax.experimental.pallas import tpu_sc as plsc

BATCH = 4096
FIELDS = 26
EMBED_DIM = 64
VOCAB = 1000000
NB = BATCH * FIELDS
NC = 2
NS = 16
NW = NC * NS
BPW = NB // NW
CHUNK = 832
NCHUNK = BPW // CHUNK
NBUF = 2

_mesh = plsc.VectorSubcoreMesh(core_axis_name="c", subcore_axis_name="s")


@functools.partial(
    pl.kernel,
    mesh=_mesh,
    out_type=jax.ShapeDtypeStruct((NB, EMBED_DIM), jnp.float32),
    scratch_types=[
        pltpu.VMEM((NCHUNK, CHUNK), jnp.int32),
        pltpu.VMEM((NBUF, CHUNK, EMBED_DIM), jnp.float32),
        pltpu.SemaphoreType.DMA,
        pltpu.SemaphoreType.DMA,
    ],
    compiler_params=pltpu.CompilerParams(use_tc_tiling_on_sc=False),
)
def _embed_gather(idx_hbm, table_hbm, out_hbm, idx_v, rows_v, gsem, ssem):
    wid = lax.axis_index("s") * NC + lax.axis_index("c")
    base = wid * BPW
    pltpu.sync_copy(idx_hbm.at[wid], idx_v)

    gathers = [None] * NCHUNK
    scatters = [None] * NCHUNK
    for g in range(min(NBUF - 1, NCHUNK)):
        gathers[g] = pltpu.async_copy(
            table_hbm.at[idx_v.at[g]], rows_v.at[g % NBUF], gsem)
    for g in range(NCHUNK):
        gathers[g].wait()
        scatters[g] = pltpu.async_copy(
            rows_v.at[g % NBUF],
            out_hbm.at[pl.ds(base + g * CHUNK, CHUNK)],
            ssem)
        nxt = g + NBUF - 1
        if nxt < NCHUNK:
            prev = nxt - NBUF
            if prev >= 0:
                scatters[prev].wait()
            gathers[nxt] = pltpu.async_copy(
                table_hbm.at[idx_v.at[nxt]], rows_v.at[nxt % NBUF], gsem)
    for g in range(max(0, NCHUNK - NBUF), NCHUNK):
        scatters[g].wait()


def kernel(feat_ids, table):
    idx = feat_ids.astype(jnp.int32).reshape(NW, NCHUNK, CHUNK)
    tflat = jax.lax.optimization_barrier(table.reshape(-1))
    out = _embed_gather(idx, tflat.reshape(VOCAB, EMBED_DIM))
    return out.reshape(BATCH, FIELDS, EMBED_DIM)


# speedup vs baseline: 1.1833x; 1.0028x over previous
import functools

import jax
import jax.numpy as jnp
from jax import lax
from jax.experimental import pallas as pl
from jax.experimental.pallas import tpu as pltpu
from jax.experimental.pallas import tpu_sc as plsc

BATCH = 4096
FIELDS = 26
EMBED_DIM = 64
VOCAB = 1000000
NB = BATCH * FIELDS
NC = 2
NS = 16
NW = NC * NS
BPW = NB // NW
CHUNK = 832
NCHUNK = BPW // CHUNK
NBUF = 2

_mesh = plsc.VectorSubcoreMesh(core_axis_name="c", subcore_axis_name="s")


@functools.partial(
    pl.kernel,
    mesh=_mesh,
    out_type=jax.ShapeDtypeStruct((NB, EMBED_DIM), jnp.float32),
    scratch_types=[
        pltpu.VMEM((NCHUNK, CHUNK), jnp.int32),
        pltpu.VMEM((NBUF, CHUNK, EMBED_DIM), jnp.float32),
        pltpu.SemaphoreType.DMA,
        pltpu.SemaphoreType.DMA,
    ],
    compiler_params=pltpu.CompilerParams(use_tc_tiling_on_sc=False),
)
def _embed_gather(idx_hbm, table_hbm, out_hbm, idx_v, rows_v, gsem, ssem):
    wid = lax.axis_index("s") * NC + lax.axis_index("c")
    base = wid * BPW
    pltpu.sync_copy(idx_hbm.at[wid], idx_v)

    gathers = [None] * NCHUNK
    scatters = [None] * NCHUNK
    for g in range(min(NBUF - 1, NCHUNK)):
        gathers[g] = pltpu.async_copy(
            table_hbm.at[idx_v.at[g]], rows_v.at[g % NBUF], gsem)
    for g in range(NCHUNK):
        gathers[g].wait()
        scatters[g] = pltpu.async_copy(
            rows_v.at[g % NBUF],
            out_hbm.at[pl.ds(base + g * CHUNK, CHUNK)],
            ssem)
        nxt = g + NBUF - 1
        if nxt < NCHUNK:
            prev = nxt - NBUF
            if prev >= 0:
                scatters[prev].wait()
            gathers[nxt] = pltpu.async_copy(
                table_hbm.at[idx_v.at[nxt]], rows_v.at[nxt % NBUF], gsem)
    for g in range(max(0, NCHUNK - NBUF), NCHUNK):
        scatters[g].wait()


def kernel(feat_ids, table):
    idx = feat_ids.astype(jnp.int32).reshape(NW, NCHUNK, CHUNK)
    out = _embed_gather(idx, table)
    return out.reshape(BATCH, FIELDS, EMBED_DIM)
